# baseline (speedup 1.0000x reference)
import functools

import jax
import jax.numpy as jnp
from jax import lax
from jax.experimental import pallas as pl
from jax.experimental.pallas import tpu as pltpu
from jax.experimental.pallas import tpu_sc as plsc

NC = 2
NS = 16
NW = NC * NS


def _gather_rows(table, idx_flat):
    R, D = table.shape
    (B,) = idx_flat.shape
    b_per_w = B // NW
    C = 64
    n_chunks = b_per_w // C
    mesh = plsc.VectorSubcoreMesh(core_axis_name="c", subcore_axis_name="s")

    @functools.partial(
        pl.kernel,
        mesh=mesh,
        out_type=jax.ShapeDtypeStruct((B, D), jnp.float32),
        scratch_types=[
            pltpu.VMEM((b_per_w,), jnp.int32),
            pltpu.VMEM((2, C, D), jnp.float32),
            pltpu.SemaphoreType.DMA,
            pltpu.SemaphoreType.DMA,
        ],
    )
    def gk(x_hbm, idx_hbm, out_hbm, idx_v, buf_v, sem0, sem1):
        wid = lax.axis_index("s") * NC + lax.axis_index("c")
        base = wid * b_per_w
        pltpu.sync_copy(idx_hbm.at[pl.ds(base, b_per_w)], idx_v)
        sems = [sem0, sem1]
        cp = pltpu.async_copy(x_hbm.at[idx_v.at[pl.ds(0, C)]], buf_v.at[0], sem0)
        copies = [cp, None]
        for c in range(n_chunks):
            copies[c % 2].wait()
            if c + 1 < n_chunks:
                copies[(c + 1) % 2] = pltpu.async_copy(
                    x_hbm.at[idx_v.at[pl.ds((c + 1) * C, C)]],
                    buf_v.at[(c + 1) % 2],
                    sems[(c + 1) % 2],
                )
            pltpu.sync_copy(buf_v.at[c % 2], out_hbm.at[pl.ds(base + c * C, C)])

    return gk(table, idx_flat)


def kernel(x, noise):
    b, n, d = x.shape
    k = max(1, n // 2)
    G = 4
    bg = b // G
    outs = []
    for g in range(G):
        ng = noise[g * bg:(g + 1) * bg]
        _, idx = lax.top_k(ng, k)
        flat_idx = (idx.astype(jnp.int32)
                    + jnp.arange(g * bg, (g + 1) * bg, dtype=jnp.int32)[:, None] * n).reshape(-1)
        outs.append(_gather_rows(x.reshape(b * n, d), flat_idx))
    return jnp.concatenate(outs).reshape(b, k, d)


# speedup vs baseline: 2.5171x; 2.5171x over previous
import functools

import jax
import jax.numpy as jnp
from jax import lax
from jax.experimental import pallas as pl
from jax.experimental.pallas import tpu as pltpu
from jax.experimental.pallas import tpu_sc as plsc

NC = 2
NS = 16
NW = NC * NS


def _sort_body(x_ref, out_ref):
    S, L = x_ref.shape
    n = 2 * L
    x = x_ref[...]
    bits = jax.lax.bitcast_convert_type(x, jnp.int32)
    key = bits ^ ((bits >> 31) & jnp.int32(0x7FFFFFFF))
    half = jax.lax.broadcasted_iota(jnp.int32, (S, L), 0) % 2
    pos = jax.lax.broadcasted_iota(jnp.int32, (S, L), 1) + half * L
    row = jax.lax.broadcasted_iota(jnp.int32, (S, L), 0) // 2
    idx = row * n + pos

    def partner(a, j, mlow):
        if j < L:
            return jnp.where(mlow, jnp.roll(a, -j, axis=1), jnp.roll(a, j, axis=1))
        return jnp.where(mlow, jnp.roll(a, -1, axis=0), jnp.roll(a, 1, axis=0))

    klev = 2
    while klev <= n:
        j = klev // 2
        while j >= 1:
            mlow = (pos & j) == 0
            pk = partner(key, j, mlow)
            pi = partner(idx, j, mlow)
            before = (key > pk) | ((key == pk) & (idx < pi))
            dirn = (pos & klev) == 0 if klev < n else (pos == pos)
            take_partner = before != (mlow == dirn)
            key = jnp.where(take_partner, pk, key)
            idx = jnp.where(take_partner, pi, idx)
            j //= 2
        klev *= 2
    out_ref[...] = idx


def _topk_indices(noise):
    b, n = noise.shape
    k = n // 2
    xs = noise.reshape(b * 2, n // 2)
    sorted_idx = pl.pallas_call(
        _sort_body,
        out_shape=jax.ShapeDtypeStruct((b * 2, n // 2), jnp.int32),
    )(xs)
    return sorted_idx.reshape(b, 2, n // 2)[:, 0, :].reshape(-1)


def _gather_rows(table, idx_flat):
    R, D = table.shape
    (B,) = idx_flat.shape
    b_per_w = B // NW
    C = 64
    n_chunks = b_per_w // C
    mesh = plsc.VectorSubcoreMesh(core_axis_name="c", subcore_axis_name="s")

    @functools.partial(
        pl.kernel,
        mesh=mesh,
        out_type=jax.ShapeDtypeStruct((B, D), jnp.float32),
        scratch_types=[
            pltpu.VMEM((b_per_w,), jnp.int32),
            pltpu.VMEM((2, C, D), jnp.float32),
            pltpu.SemaphoreType.DMA,
            pltpu.SemaphoreType.DMA,
        ],
    )
    def gk(x_hbm, idx_hbm, out_hbm, idx_v, buf_v, sem0, sem1):
        wid = lax.axis_index("s") * NC + lax.axis_index("c")
        base = wid * b_per_w
        pltpu.sync_copy(idx_hbm.at[pl.ds(base, b_per_w)], idx_v)
        sems = [sem0, sem1]
        cp = pltpu.async_copy(x_hbm.at[idx_v.at[pl.ds(0, C)]], buf_v.at[0], sem0)
        copies = [cp, None]
        for c in range(n_chunks):
            copies[c % 2].wait()
            if c + 1 < n_chunks:
                copies[(c + 1) % 2] = pltpu.async_copy(
                    x_hbm.at[idx_v.at[pl.ds((c + 1) * C, C)]],
                    buf_v.at[(c + 1) % 2],
                    sems[(c + 1) % 2],
                )
            pltpu.sync_copy(buf_v.at[c % 2], out_hbm.at[pl.ds(base + c * C, C)])

    return gk(table, idx_flat)


def kernel(x, noise):
    b, n, d = x.shape
    k = max(1, n // 2)
    flat_idx = _topk_indices(noise)
    out = _gather_rows(x.reshape(b * n, d), flat_idx)
    return out.reshape(b, k, d)


# speedup vs baseline: 2.5759x; 1.0234x over previous
import functools

import jax
import jax.numpy as jnp
from jax import lax
from jax.experimental import pallas as pl
from jax.experimental.pallas import tpu as pltpu
from jax.experimental.pallas import tpu_sc as plsc

NC = 2
NS = 16
NW = NC * NS


def _sort_body(x_ref, out_ref):
    b, n = x_ref.shape
    L = n // 2
    S = 2 * b
    x = jnp.concatenate([x_ref[:, :L], x_ref[:, L:]], axis=0)
    bits = jax.lax.bitcast_convert_type(x, jnp.int32)
    key = bits ^ ((bits >> 31) & jnp.int32(0x7FFFFFFF))
    half = jax.lax.broadcasted_iota(jnp.int32, (S, L), 0) // b
    pos = jax.lax.broadcasted_iota(jnp.int32, (S, L), 1) + half * L
    row = jax.lax.broadcasted_iota(jnp.int32, (S, L), 0) % b
    idx = row * n + pos

    def partner(a, j, mlow):
        if j < L:
            return jnp.where(mlow, jnp.roll(a, -j, axis=1), jnp.roll(a, j, axis=1))
        return jnp.roll(a, b, axis=0)

    klev = 2
    while klev <= n:
        j = klev // 2
        while j >= 1:
            mlow = (pos & j) == 0
            pk = partner(key, j, mlow)
            pi = partner(idx, j, mlow)
            before = (key > pk) | ((key == pk) & (idx < pi))
            dirn = (pos & klev) == 0 if klev < n else (pos == pos)
            take_partner = before != (mlow == dirn)
            key = jnp.where(take_partner, pk, key)
            idx = jnp.where(take_partner, pi, idx)
            j //= 2
        klev *= 2
    out_ref[...] = idx[0:b, :]


def _topk_indices(noise):
    b, n = noise.shape
    k = n // 2
    sorted_idx = pl.pallas_call(
        _sort_body,
        out_shape=jax.ShapeDtypeStruct((b, k), jnp.int32),
    )(noise)
    return sorted_idx.reshape(-1)


def _gather_rows(table, idx_flat):
    R, D = table.shape
    (B,) = idx_flat.shape
    b_per_w = B // NW
    C = 64
    n_chunks = b_per_w // C
    mesh = plsc.VectorSubcoreMesh(core_axis_name="c", subcore_axis_name="s")

    @functools.partial(
        pl.kernel,
        mesh=mesh,
        out_type=jax.ShapeDtypeStruct((B, D), jnp.float32),
        scratch_types=[
            pltpu.VMEM((b_per_w,), jnp.int32),
            pltpu.VMEM((2, C, D), jnp.float32),
            pltpu.SemaphoreType.DMA,
            pltpu.SemaphoreType.DMA,
        ],
    )
    def gk(x_hbm, idx_hbm, out_hbm, idx_v, buf_v, sem0, sem1):
        wid = lax.axis_index("s") * NC + lax.axis_index("c")
        base = wid * b_per_w
        pltpu.sync_copy(idx_hbm.at[pl.ds(base, b_per_w)], idx_v)
        sems = [sem0, sem1]
        cp = pltpu.async_copy(x_hbm.at[idx_v.at[pl.ds(0, C)]], buf_v.at[0], sem0)
        copies = [cp, None]
        for c in range(n_chunks):
            copies[c % 2].wait()
            if c + 1 < n_chunks:
                copies[(c + 1) % 2] = pltpu.async_copy(
                    x_hbm.at[idx_v.at[pl.ds((c + 1) * C, C)]],
                    buf_v.at[(c + 1) % 2],
                    sems[(c + 1) % 2],
                )
            pltpu.sync_copy(buf_v.at[c % 2], out_hbm.at[pl.ds(base + c * C, C)])

    return gk(table, idx_flat)


def kernel(x, noise):
    b, n, d = x.shape
    k = max(1, n // 2)
    flat_idx = _topk_indices(noise)
    out = _gather_rows(x.reshape(b * n, d), flat_idx)
    return out.reshape(b, k, d)
